# Initial kernel scaffold; baseline (speedup 1.0000x reference)
import jax
import jax.numpy as jnp
from jax.experimental import pallas as pl


def kernel(x, edge_index, W1, b1, W2, b2, W3, b3, W4, b4, W5, b5):
    raise NotImplementedError("write your pallas kernel here")


# speedup vs baseline: 11.9398x; 11.9398x over previous
import jax
import jax.numpy as jnp
from jax import lax
from jax.experimental import pallas as pl
from jax.experimental.pallas import tpu as pltpu
from jax.experimental.pallas import tpu_sc as plsc

N = 10000
E = 160000
NP = 10240
EP = 163840
CW = 128
NCH = EP // CW
NSC = 2
NTL = 16
CPT = NCH // (NSC * NTL)
RPT = NP // NTL
BR = 2560

import functools as _functools


@_functools.cache
def _sc_mesh():
    return plsc.VectorSubcoreMesh(core_axis_name="c", subcore_axis_name="s",
                                  num_cores=NSC, num_subcores=NTL)


def _deg_body(dst_hbm, out_hbm, idx_v, buf_v, acc_sh, sem):
    c = lax.axis_index("c")
    s = lax.axis_index("s")
    wid = c * NTL + s
    pltpu.sync_copy(dst_hbm.at[pl.ds(wid * CPT, CPT)], idx_v)
    for i in range(CW):
        buf_v[i, :] = jnp.zeros((16,), jnp.float32)
    for k in range(RPT // CW):
        pltpu.sync_copy(buf_v, acc_sh.at[pl.ds(s * RPT + k * CW, CW)])
    for i in range(CW):
        buf_v[i, :] = jnp.full((16,), 1.0, jnp.float32)
    plsc.subcore_barrier()

    def chunk(j, carry):
        pltpu.sync_copy(buf_v, acc_sh.at[idx_v.at[j]], add=True)
        return carry

    lax.fori_loop(0, CPT, chunk, 0)
    plsc.subcore_barrier()
    pltpu.sync_copy(acc_sh.at[pl.ds(s * RPT, RPT)],
                    out_hbm.at[c, pl.ds(s * RPT, RPT)])


def _sc_degree(dstp):
    return pl.kernel(
        _deg_body,
        out_type=jax.ShapeDtypeStruct((NSC, NP, 16), jnp.float32),
        mesh=_sc_mesh(),
        scratch_types=[
            pltpu.VMEM((CPT, CW), jnp.int32),
            pltpu.VMEM((CW, 16), jnp.float32),
            pltpu.VMEM_SHARED((NP, 16), jnp.float32),
            pltpu.SemaphoreType.DMA,
        ],
        compiler_params=pltpu.CompilerParams(use_tc_tiling_on_sc=False),
    )(dstp)


def _agg_body(hs_hbm, src_hbm, dst_hbm, out_hbm, sidx_v, didx_v, row_v,
              acc_sh, sem, *, F):
    c = lax.axis_index("c")
    s = lax.axis_index("s")
    wid = c * NTL + s
    pltpu.sync_copy(src_hbm.at[pl.ds(wid * CPT, CPT)], sidx_v)
    pltpu.sync_copy(dst_hbm.at[pl.ds(wid * CPT, CPT)], didx_v)
    for i in range(CW):
        for k in range(F // 16):
            row_v[i, pl.ds(k * 16, 16)] = jnp.zeros((16,), jnp.float32)
    for k in range(RPT // CW):
        pltpu.sync_copy(row_v, acc_sh.at[pl.ds(s * RPT + k * CW, CW)])
    plsc.subcore_barrier()

    def chunk(j, carry):
        pltpu.async_copy(hs_hbm.at[sidx_v.at[j]], row_v, sem).wait()
        pltpu.sync_copy(row_v, acc_sh.at[didx_v.at[j]], add=True)
        return carry

    lax.fori_loop(0, CPT, chunk, 0)
    plsc.subcore_barrier()
    pltpu.sync_copy(acc_sh.at[pl.ds(s * RPT, RPT)],
                    out_hbm.at[c, pl.ds(s * RPT, RPT)])


def _sc_aggregate(hs, srcp, dstp):
    F = hs.shape[1]

    def body(*refs):
        _agg_body(*refs, F=F)

    return pl.kernel(
        body,
        out_type=jax.ShapeDtypeStruct((NSC, NP, F), jnp.float32),
        mesh=_sc_mesh(),
        scratch_types=[
            pltpu.VMEM((CPT, CW), jnp.int32),
            pltpu.VMEM((CPT, CW), jnp.int32),
            pltpu.VMEM((CW, F), jnp.float32),
            pltpu.VMEM_SHARED((NP, F), jnp.float32),
            pltpu.SemaphoreType.DMA,
        ],
        compiler_params=pltpu.CompilerParams(use_tc_tiling_on_sc=False),
    )(hs, srcp, dstp)


def _prep_body(x_ref, c0_ref, c1_ref, w_ref, dinv_ref, hs_ref):
    i = pl.program_id(0)
    deg = c0_ref[...] + c1_ref[...] + 1.0
    dinv = lax.rsqrt(deg)
    rows = i * BR + lax.broadcasted_iota(jnp.int32, (BR, 16), 0)
    dinv = jnp.where(rows < N, dinv, 0.0)
    dinv_ref[...] = dinv
    h = jnp.dot(x_ref[...], w_ref[...], preferred_element_type=jnp.float32)
    hs_ref[...] = h * dinv[:, :1]


def _tc_prep(xp, cnt, w1p):
    f = w1p.shape[1]
    return pl.pallas_call(
        _prep_body,
        grid=(NP // BR,),
        in_specs=[
            pl.BlockSpec((BR, 256), lambda i: (i, 0)),
            pl.BlockSpec((BR, 16), lambda i: (i, 0)),
            pl.BlockSpec((BR, 16), lambda i: (i, 0)),
            pl.BlockSpec((256, f), lambda i: (0, 0)),
        ],
        out_specs=[
            pl.BlockSpec((BR, 16), lambda i: (i, 0)),
            pl.BlockSpec((BR, f), lambda i: (i, 0)),
        ],
        out_shape=[
            jax.ShapeDtypeStruct((NP, 16), jnp.float32),
            jax.ShapeDtypeStruct((NP, f), jnp.float32),
        ],
    )(xp, cnt[0], cnt[1], w1p)


def _mid_body(p_ref, hs_ref, dinv_ref, b_ref, w_ref, hso_ref, emb_ref=None):
    d = dinv_ref[...][:, :1]
    o = d * (p_ref[0] + p_ref[1] + hs_ref[...]) + b_ref[...]
    xn = jnp.maximum(o, 0.0)
    if emb_ref is not None:
        emb_ref[...] = xn
    hso_ref[...] = jnp.dot(xn, w_ref[...],
                           preferred_element_type=jnp.float32) * d


def _tc_mid(p, hs, dinv, bp, wp, emit_emb=False):
    fi, fo = wp.shape
    outs = [jax.ShapeDtypeStruct((NP, fo), jnp.float32)]
    out_specs = [pl.BlockSpec((BR, fo), lambda i: (i, 0))]
    if emit_emb:
        outs.append(jax.ShapeDtypeStruct((NP, fi), jnp.float32))
        out_specs.append(pl.BlockSpec((BR, fi), lambda i: (i, 0)))
    res = pl.pallas_call(
        _mid_body,
        grid=(NP // BR,),
        in_specs=[
            pl.BlockSpec((NSC, BR, fi), lambda i: (0, i, 0)),
            pl.BlockSpec((BR, fi), lambda i: (i, 0)),
            pl.BlockSpec((BR, 16), lambda i: (i, 0)),
            pl.BlockSpec((1, fi), lambda i: (0, 0)),
            pl.BlockSpec((fi, fo), lambda i: (0, 0)),
        ],
        out_specs=out_specs,
        out_shape=outs,
    )(p, hs, dinv, bp, wp)
    return res if emit_emb else res[0]


def _fin_body(p_ref, hs_ref, dinv_ref, b_ref, out_ref, logp_ref):
    d = dinv_ref[...][:, :1]
    o = d * (p_ref[0] + p_ref[1] + hs_ref[...]) + b_ref[...]
    col = lax.broadcasted_iota(jnp.int32, (BR, 16), 1)
    valid = col < 10
    om = jnp.where(valid, o, jnp.float32(-1e30))
    m = jnp.max(om, axis=1, keepdims=True)
    ex = jnp.where(valid, jnp.exp(o - m), 0.0)
    lse = jnp.log(jnp.sum(ex, axis=1, keepdims=True)) + m
    out_ref[...] = o
    logp_ref[...] = o - lse


def _tc_final(p, hs, dinv, bp):
    return pl.pallas_call(
        _fin_body,
        grid=(NP // BR,),
        in_specs=[
            pl.BlockSpec((NSC, BR, 16), lambda i: (0, i, 0)),
            pl.BlockSpec((BR, 16), lambda i: (i, 0)),
            pl.BlockSpec((BR, 16), lambda i: (i, 0)),
            pl.BlockSpec((1, 16), lambda i: (0, 0)),
        ],
        out_specs=[
            pl.BlockSpec((BR, 16), lambda i: (i, 0)),
            pl.BlockSpec((BR, 16), lambda i: (i, 0)),
        ],
        out_shape=[
            jax.ShapeDtypeStruct((NP, 16), jnp.float32),
            jax.ShapeDtypeStruct((NP, 16), jnp.float32),
        ],
    )(p, hs, dinv, bp)


def _pad2(w, r, c):
    return jnp.pad(w, ((0, r - w.shape[0]), (0, c - w.shape[1])))


def kernel(x, edge_index, W1, b1, W2, b2, W3, b3, W4, b4, W5, b5):
    pad = jnp.full((EP - E,), N, jnp.int32)
    srcp = jnp.concatenate([edge_index[0], pad]).reshape(NCH, CW)
    dstp = jnp.concatenate([edge_index[1], pad]).reshape(NCH, CW)
    xp = jnp.pad(x, ((0, NP - N), (0, 0)))

    w1 = _pad2(W1, 256, 64)
    w2 = _pad2(W2, 64, 64)
    w3 = _pad2(W3, 64, 32)
    w4 = _pad2(W4, 32, 16)
    w5 = _pad2(W5, 16, 16)
    b1p = jnp.pad(b1, (0, 64 - 60)).reshape(1, 64)
    b2p = jnp.pad(b2, (0, 64 - 60)).reshape(1, 64)
    b3p = jnp.pad(b3, (0, 32 - 30)).reshape(1, 32)
    b4p = b4.reshape(1, 16)
    b5p = jnp.pad(b5, (0, 16 - 10)).reshape(1, 16)

    cnt = _sc_degree(dstp)
    dinv, hs1 = _tc_prep(xp, cnt, w1)

    p1 = _sc_aggregate(hs1, srcp, dstp)
    hs2 = _tc_mid(p1, hs1, dinv, b1p, w2)
    p2 = _sc_aggregate(hs2, srcp, dstp)
    hs3 = _tc_mid(p2, hs2, dinv, b2p, w3)
    p3 = _sc_aggregate(hs3, srcp, dstp)
    hs4 = _tc_mid(p3, hs3, dinv, b3p, w4)
    p4 = _sc_aggregate(hs4, srcp, dstp)
    hs5, emb = _tc_mid(p4, hs4, dinv, b4p, w5, emit_emb=True)
    p5 = _sc_aggregate(hs5, srcp, dstp)
    out_f, logp_f = _tc_final(p5, hs5, dinv, b5p)

    return out_f[:N, :10], logp_f[:N, :10], emb[:N, :]


# speedup vs baseline: 15.1256x; 1.2668x over previous
import jax
import jax.numpy as jnp
from jax import lax
from jax.experimental import pallas as pl
from jax.experimental.pallas import tpu as pltpu
from jax.experimental.pallas import tpu_sc as plsc

N = 10000
E = 160000
NP = 10240
EP = 163840
CW = 128
NCH = EP // CW
NSC = 2
NTL = 16
CPT = NCH // (NSC * NTL)
RPT = NP // NTL
BR = 2560

import functools as _functools


@_functools.cache
def _sc_mesh():
    return plsc.VectorSubcoreMesh(core_axis_name="c", subcore_axis_name="s",
                                  num_cores=NSC, num_subcores=NTL)


def _deg_body(dst_hbm, out_hbm, idx_v, buf_v, acc_sh, sem):
    c = lax.axis_index("c")
    s = lax.axis_index("s")
    wid = c * NTL + s
    pltpu.sync_copy(dst_hbm.at[pl.ds(wid * CPT, CPT)], idx_v)
    for i in range(CW):
        buf_v[i, :] = jnp.zeros((16,), jnp.float32)
    for k in range(RPT // CW):
        pltpu.sync_copy(buf_v, acc_sh.at[pl.ds(s * RPT + k * CW, CW)])
    for i in range(CW):
        buf_v[i, :] = jnp.full((16,), 1.0, jnp.float32)
    plsc.subcore_barrier()

    def chunk(j, carry):
        pltpu.sync_copy(buf_v, acc_sh.at[idx_v.at[j]], add=True)
        return carry

    lax.fori_loop(0, CPT, chunk, 0)
    plsc.subcore_barrier()
    pltpu.sync_copy(acc_sh.at[pl.ds(s * RPT, RPT)],
                    out_hbm.at[c, pl.ds(s * RPT, RPT)])


def _sc_degree(dstp):
    return pl.kernel(
        _deg_body,
        out_type=jax.ShapeDtypeStruct((NSC, NP, 16), jnp.float32),
        mesh=_sc_mesh(),
        scratch_types=[
            pltpu.VMEM((CPT, CW), jnp.int32),
            pltpu.VMEM((CW, 16), jnp.float32),
            pltpu.VMEM_SHARED((NP, 16), jnp.float32),
            pltpu.SemaphoreType.DMA,
        ],
        compiler_params=pltpu.CompilerParams(use_tc_tiling_on_sc=False),
    )(dstp)


NBUF = 4


def _agg_body(hs_hbm, src_hbm, dst_hbm, out_hbm, sidx_v, didx_v, row_v,
              acc_sh, sems, *, F):
    c = lax.axis_index("c")
    s = lax.axis_index("s")
    wid = c * NTL + s
    pltpu.sync_copy(src_hbm.at[pl.ds(wid * CPT, CPT)], sidx_v)
    pltpu.sync_copy(dst_hbm.at[pl.ds(wid * CPT, CPT)], didx_v)
    for i in range(CW):
        for k in range(F // 16):
            row_v[0, i, pl.ds(k * 16, 16)] = jnp.zeros((16,), jnp.float32)
    for k in range(RPT // CW):
        pltpu.sync_copy(row_v.at[0], acc_sh.at[pl.ds(s * RPT + k * CW, CW)])
    plsc.subcore_barrier()

    for b in range(NBUF):
        pltpu.async_copy(hs_hbm.at[sidx_v.at[b]], row_v.at[b], sems.at[b])

    def ring(t, carry):
        for b in range(NBUF):
            j = t + b
            pltpu.make_async_copy(hs_hbm.at[sidx_v.at[j]], row_v.at[b],
                                  sems.at[b]).wait()
            pltpu.sync_copy(row_v.at[b], acc_sh.at[didx_v.at[j]], add=True)

            @pl.when(j + NBUF < CPT)
            def _():
                pltpu.async_copy(hs_hbm.at[sidx_v.at[j + NBUF]], row_v.at[b],
                                 sems.at[b])
        return carry

    lax.fori_loop(0, CPT // NBUF, lambda t, cy: ring(t * NBUF, cy), 0)
    plsc.subcore_barrier()
    pltpu.sync_copy(acc_sh.at[pl.ds(s * RPT, RPT)],
                    out_hbm.at[c, pl.ds(s * RPT, RPT)])


def _sc_aggregate(hs, srcp, dstp):
    F = hs.shape[1]

    def body(*refs):
        _agg_body(*refs, F=F)

    return pl.kernel(
        body,
        out_type=jax.ShapeDtypeStruct((NSC, NP, F), jnp.float32),
        mesh=_sc_mesh(),
        scratch_types=[
            pltpu.VMEM((CPT, CW), jnp.int32),
            pltpu.VMEM((CPT, CW), jnp.int32),
            pltpu.VMEM((NBUF, CW, F), jnp.float32),
            pltpu.VMEM_SHARED((NP, F), jnp.float32),
            pltpu.SemaphoreType.DMA((NBUF,)),
        ],
        compiler_params=pltpu.CompilerParams(use_tc_tiling_on_sc=False),
    )(hs, srcp, dstp)


def _mm1_body(x_ref, w_ref, h_ref):
    h_ref[...] = jnp.dot(x_ref[...], w_ref[...],
                         preferred_element_type=jnp.float32)


def _tc_mm1(xp, w1p):
    f = w1p.shape[1]
    return pl.pallas_call(
        _mm1_body,
        grid=(NP // BR,),
        in_specs=[
            pl.BlockSpec((BR, 256), lambda i: (i, 0)),
            pl.BlockSpec((256, f), lambda i: (0, 0)),
        ],
        out_specs=pl.BlockSpec((BR, f), lambda i: (i, 0)),
        out_shape=jax.ShapeDtypeStruct((NP, f), jnp.float32),
    )(xp, w1p)


def _scale_body(h_ref, c0_ref, c1_ref, dinv_ref, hs_ref):
    i = pl.program_id(0)
    deg = c0_ref[...] + c1_ref[...] + 1.0
    dinv = lax.rsqrt(deg)
    rows = i * BR + lax.broadcasted_iota(jnp.int32, (BR, 16), 0)
    dinv = jnp.where(rows < N, dinv, 0.0)
    dinv_ref[...] = dinv
    hs_ref[...] = h_ref[...] * dinv[:, :1]


def _tc_prep(xp, cnt, w1p):
    f = w1p.shape[1]
    h1 = _tc_mm1(xp, w1p)
    return pl.pallas_call(
        _scale_body,
        grid=(NP // BR,),
        in_specs=[
            pl.BlockSpec((BR, f), lambda i: (i, 0)),
            pl.BlockSpec((BR, 16), lambda i: (i, 0)),
            pl.BlockSpec((BR, 16), lambda i: (i, 0)),
        ],
        out_specs=[
            pl.BlockSpec((BR, 16), lambda i: (i, 0)),
            pl.BlockSpec((BR, f), lambda i: (i, 0)),
        ],
        out_shape=[
            jax.ShapeDtypeStruct((NP, 16), jnp.float32),
            jax.ShapeDtypeStruct((NP, f), jnp.float32),
        ],
    )(h1, cnt[0], cnt[1])


def _mid_body(p_ref, hs_ref, dinv_ref, b_ref, w_ref, hso_ref, emb_ref=None):
    d = dinv_ref[...][:, :1]
    o = d * (p_ref[0] + p_ref[1] + hs_ref[...]) + b_ref[...]
    xn = jnp.maximum(o, 0.0)
    if emb_ref is not None:
        emb_ref[...] = xn
    hso_ref[...] = jnp.dot(xn, w_ref[...],
                           preferred_element_type=jnp.float32) * d


def _tc_mid(p, hs, dinv, bp, wp, emit_emb=False):
    fi, fo = wp.shape
    outs = [jax.ShapeDtypeStruct((NP, fo), jnp.float32)]
    out_specs = [pl.BlockSpec((BR, fo), lambda i: (i, 0))]
    if emit_emb:
        outs.append(jax.ShapeDtypeStruct((NP, fi), jnp.float32))
        out_specs.append(pl.BlockSpec((BR, fi), lambda i: (i, 0)))
    res = pl.pallas_call(
        _mid_body,
        grid=(NP // BR,),
        in_specs=[
            pl.BlockSpec((NSC, BR, fi), lambda i: (0, i, 0)),
            pl.BlockSpec((BR, fi), lambda i: (i, 0)),
            pl.BlockSpec((BR, 16), lambda i: (i, 0)),
            pl.BlockSpec((1, fi), lambda i: (0, 0)),
            pl.BlockSpec((fi, fo), lambda i: (0, 0)),
        ],
        out_specs=out_specs,
        out_shape=outs,
    )(p, hs, dinv, bp, wp)
    return res if emit_emb else res[0]


def _fin_body(p_ref, hs_ref, dinv_ref, b_ref, out_ref, logp_ref):
    d = dinv_ref[...][:, :1]
    o = d * (p_ref[0] + p_ref[1] + hs_ref[...]) + b_ref[...]
    col = lax.broadcasted_iota(jnp.int32, (BR, 16), 1)
    valid = col < 10
    om = jnp.where(valid, o, jnp.float32(-1e30))
    m = jnp.max(om, axis=1, keepdims=True)
    ex = jnp.where(valid, jnp.exp(o - m), 0.0)
    lse = jnp.log(jnp.sum(ex, axis=1, keepdims=True)) + m
    out_ref[...] = o
    logp_ref[...] = o - lse


def _tc_final(p, hs, dinv, bp):
    return pl.pallas_call(
        _fin_body,
        grid=(NP // BR,),
        in_specs=[
            pl.BlockSpec((NSC, BR, 16), lambda i: (0, i, 0)),
            pl.BlockSpec((BR, 16), lambda i: (i, 0)),
            pl.BlockSpec((BR, 16), lambda i: (i, 0)),
            pl.BlockSpec((1, 16), lambda i: (0, 0)),
        ],
        out_specs=[
            pl.BlockSpec((BR, 16), lambda i: (i, 0)),
            pl.BlockSpec((BR, 16), lambda i: (i, 0)),
        ],
        out_shape=[
            jax.ShapeDtypeStruct((NP, 16), jnp.float32),
            jax.ShapeDtypeStruct((NP, 16), jnp.float32),
        ],
    )(p, hs, dinv, bp)


def _pad2(w, r, c):
    return jnp.pad(w, ((0, r - w.shape[0]), (0, c - w.shape[1])))


def kernel(x, edge_index, W1, b1, W2, b2, W3, b3, W4, b4, W5, b5):
    pad = jnp.full((EP - E,), N, jnp.int32)
    srcp = jnp.concatenate([edge_index[0], pad]).reshape(NCH, CW)
    dstp = jnp.concatenate([edge_index[1], pad]).reshape(NCH, CW)
    xp = jnp.pad(x, ((0, NP - N), (0, 0)))

    w1 = _pad2(W1, 256, 64)
    w2 = _pad2(W2, 64, 64)
    w3 = _pad2(W3, 64, 32)
    w4 = _pad2(W4, 32, 16)
    w5 = _pad2(W5, 16, 16)
    b1p = jnp.pad(b1, (0, 64 - 60)).reshape(1, 64)
    b2p = jnp.pad(b2, (0, 64 - 60)).reshape(1, 64)
    b3p = jnp.pad(b3, (0, 32 - 30)).reshape(1, 32)
    b4p = b4.reshape(1, 16)
    b5p = jnp.pad(b5, (0, 16 - 10)).reshape(1, 16)

    cnt = _sc_degree(dstp)
    dinv, hs1 = _tc_prep(xp, cnt, w1)

    p1 = _sc_aggregate(hs1, srcp, dstp)
    hs2 = _tc_mid(p1, hs1, dinv, b1p, w2)
    p2 = _sc_aggregate(hs2, srcp, dstp)
    hs3 = _tc_mid(p2, hs2, dinv, b2p, w3)
    p3 = _sc_aggregate(hs3, srcp, dstp)
    hs4 = _tc_mid(p3, hs3, dinv, b3p, w4)
    p4 = _sc_aggregate(hs4, srcp, dstp)
    hs5, emb = _tc_mid(p4, hs4, dinv, b4p, w5, emit_emb=True)
    p5 = _sc_aggregate(hs5, srcp, dstp)
    out_f, logp_f = _tc_final(p5, hs5, dinv, b5p)

    return out_f[:N, :10], logp_f[:N, :10], emb[:N, :]


# speedup vs baseline: 25.8290x; 1.7076x over previous
import jax
import jax.numpy as jnp
from jax import lax
from jax.experimental import pallas as pl
from jax.experimental.pallas import tpu as pltpu
from jax.experimental.pallas import tpu_sc as plsc

N = 10000
E = 160000
NP = 10240
EP = 163840
CW = 128
NCH = EP // CW
NSC = 2
NTL = 16
CPT = NCH // (NSC * NTL)
RPT = NP // NTL
BR = 2560

import functools as _functools


@_functools.cache
def _sc_mesh():
    return plsc.VectorSubcoreMesh(core_axis_name="c", subcore_axis_name="s",
                                  num_cores=NSC, num_subcores=NTL)


def _deg_body(dst_hbm, out_hbm, idx_v, buf_v, acc_sh, sem):
    c = lax.axis_index("c")
    s = lax.axis_index("s")
    wid = c * NTL + s
    pltpu.sync_copy(dst_hbm.at[pl.ds(wid * CPT, CPT)], idx_v)
    for i in range(CW):
        buf_v[i, :] = jnp.zeros((16,), jnp.float32)
    for k in range(RPT // CW):
        pltpu.sync_copy(buf_v, acc_sh.at[pl.ds(s * RPT + k * CW, CW)])
    for i in range(CW):
        buf_v[i, :] = jnp.full((16,), 1.0, jnp.float32)
    plsc.subcore_barrier()

    def chunk(j, carry):
        pltpu.sync_copy(buf_v, acc_sh.at[idx_v.at[j]], add=True)
        return carry

    lax.fori_loop(0, CPT, chunk, 0)
    plsc.subcore_barrier()
    pltpu.sync_copy(acc_sh.at[pl.ds(s * RPT, RPT)],
                    out_hbm.at[c, pl.ds(s * RPT, RPT)])


def _sc_degree(dstp):
    return pl.kernel(
        _deg_body,
        out_type=jax.ShapeDtypeStruct((NSC, NP, 16), jnp.float32),
        mesh=_sc_mesh(),
        scratch_types=[
            pltpu.VMEM((CPT, CW), jnp.int32),
            pltpu.VMEM((CW, 16), jnp.float32),
            pltpu.VMEM_SHARED((NP, 16), jnp.float32),
            pltpu.SemaphoreType.DMA,
        ],
        compiler_params=pltpu.CompilerParams(use_tc_tiling_on_sc=False),
    )(dstp)


NBUF = 4


def _agg_body(hs_hbm, src_hbm, dst_hbm, out_hbm, sidx_v, didx_v, row_v,
              acc_sh, sems, *, F):
    c = lax.axis_index("c")
    s = lax.axis_index("s")
    wid = c * NTL + s
    pltpu.sync_copy(src_hbm.at[pl.ds(wid * CPT, CPT)], sidx_v)
    pltpu.sync_copy(dst_hbm.at[pl.ds(wid * CPT, CPT)], didx_v)
    for i in range(CW):
        for k in range(F // 16):
            row_v[0, i, pl.ds(k * 16, 16)] = jnp.zeros((16,), jnp.float32)
    for k in range(RPT // CW):
        pltpu.sync_copy(row_v.at[0], acc_sh.at[pl.ds(s * RPT + k * CW, CW)])
    plsc.subcore_barrier()

    for b in range(NBUF):
        pltpu.async_copy(hs_hbm.at[sidx_v.at[b]], row_v.at[b], sems.at[b])

    def ring(t, carry):
        for b in range(NBUF):
            j = t + b
            pltpu.make_async_copy(hs_hbm.at[sidx_v.at[j]], row_v.at[b],
                                  sems.at[b]).wait()
            pltpu.sync_copy(row_v.at[b], acc_sh.at[didx_v.at[j]], add=True)

            @pl.when(j + NBUF < CPT)
            def _():
                pltpu.async_copy(hs_hbm.at[sidx_v.at[j + NBUF]], row_v.at[b],
                                 sems.at[b])
        return carry

    lax.fori_loop(0, CPT // NBUF, lambda t, cy: ring(t * NBUF, cy), 0)
    plsc.subcore_barrier()
    pltpu.sync_copy(acc_sh.at[pl.ds(s * RPT, RPT)],
                    out_hbm.at[c, pl.ds(s * RPT, RPT)])


def _sc_aggregate(hs, srcp, dstp):
    F = hs.shape[1]

    def body(*refs):
        _agg_body(*refs, F=F)

    return pl.kernel(
        body,
        out_type=jax.ShapeDtypeStruct((NSC, NP, F), jnp.float32),
        mesh=_sc_mesh(),
        scratch_types=[
            pltpu.VMEM((CPT, CW), jnp.int32),
            pltpu.VMEM((CPT, CW), jnp.int32),
            pltpu.VMEM((NBUF, CW, F), jnp.float32),
            pltpu.VMEM_SHARED((NP, F), jnp.float32),
            pltpu.SemaphoreType.DMA((NBUF,)),
        ],
        compiler_params=pltpu.CompilerParams(use_tc_tiling_on_sc=False),
    )(hs, srcp, dstp)


def _mm1_body(x_ref, w_ref, h_ref):
    h_ref[...] = jnp.dot(x_ref[...], w_ref[...],
                         preferred_element_type=jnp.float32)


def _tc_mm1(xp, w1p):
    f = w1p.shape[1]
    return pl.pallas_call(
        _mm1_body,
        grid=(NP // BR,),
        in_specs=[
            pl.BlockSpec((BR, 256), lambda i: (i, 0)),
            pl.BlockSpec((256, f), lambda i: (0, 0)),
        ],
        out_specs=pl.BlockSpec((BR, f), lambda i: (i, 0)),
        out_shape=jax.ShapeDtypeStruct((NP, f), jnp.float32),
    )(xp, w1p)


def _scale_body(h_ref, c0_ref, c1_ref, dinv_ref, hs_ref):
    i = pl.program_id(0)
    deg = c0_ref[...] + c1_ref[...] + 1.0
    dinv = lax.rsqrt(deg)
    rows = i * BR + lax.broadcasted_iota(jnp.int32, (BR, 16), 0)
    dinv = jnp.where(rows < N, dinv, 0.0)
    dinv_ref[...] = dinv
    hs_ref[...] = h_ref[...] * dinv[:, :1]


def _tc_prep(xp, cnt, w1p):
    f = w1p.shape[1]
    h1 = _tc_mm1(xp, w1p)
    return pl.pallas_call(
        _scale_body,
        grid=(NP // BR,),
        in_specs=[
            pl.BlockSpec((BR, f), lambda i: (i, 0)),
            pl.BlockSpec((BR, 16), lambda i: (i, 0)),
            pl.BlockSpec((BR, 16), lambda i: (i, 0)),
        ],
        out_specs=[
            pl.BlockSpec((BR, 16), lambda i: (i, 0)),
            pl.BlockSpec((BR, f), lambda i: (i, 0)),
        ],
        out_shape=[
            jax.ShapeDtypeStruct((NP, 16), jnp.float32),
            jax.ShapeDtypeStruct((NP, f), jnp.float32),
        ],
    )(h1, cnt[0], cnt[1])


def _mid_body(p_ref, hs_ref, dinv_ref, b_ref, w_ref, hso_ref, emb_ref=None):
    d = dinv_ref[...][:, :1]
    o = d * (p_ref[0] + p_ref[1] + hs_ref[...]) + b_ref[...]
    xn = jnp.maximum(o, 0.0)
    if emb_ref is not None:
        emb_ref[...] = xn
    hso_ref[...] = jnp.dot(xn, w_ref[...],
                           preferred_element_type=jnp.float32) * d


def _tc_mid(p, hs, dinv, bp, wp, emit_emb=False):
    fi, fo = wp.shape
    outs = [jax.ShapeDtypeStruct((NP, fo), jnp.float32)]
    out_specs = [pl.BlockSpec((BR, fo), lambda i: (i, 0))]
    if emit_emb:
        outs.append(jax.ShapeDtypeStruct((NP, fi), jnp.float32))
        out_specs.append(pl.BlockSpec((BR, fi), lambda i: (i, 0)))
    res = pl.pallas_call(
        _mid_body,
        grid=(NP // BR,),
        in_specs=[
            pl.BlockSpec((NSC, BR, fi), lambda i: (0, i, 0)),
            pl.BlockSpec((BR, fi), lambda i: (i, 0)),
            pl.BlockSpec((BR, 16), lambda i: (i, 0)),
            pl.BlockSpec((1, fi), lambda i: (0, 0)),
            pl.BlockSpec((fi, fo), lambda i: (0, 0)),
        ],
        out_specs=out_specs,
        out_shape=outs,
    )(p, hs, dinv, bp, wp)
    return res if emit_emb else res[0]


def _fin_body(p_ref, hs_ref, dinv_ref, b_ref, out_ref, logp_ref):
    d = dinv_ref[...][:, :1]
    o = d * (p_ref[0] + p_ref[1] + hs_ref[...]) + b_ref[...]
    col = lax.broadcasted_iota(jnp.int32, (BR, 16), 1)
    valid = col < 10
    om = jnp.where(valid, o, jnp.float32(-1e30))
    m = jnp.max(om, axis=1, keepdims=True)
    ex = jnp.where(valid, jnp.exp(o - m), 0.0)
    lse = jnp.log(jnp.sum(ex, axis=1, keepdims=True)) + m
    out_ref[...] = o
    logp_ref[...] = o - lse


def _tc_final(p, hs, dinv, bp):
    return pl.pallas_call(
        _fin_body,
        grid=(NP // BR,),
        in_specs=[
            pl.BlockSpec((NSC, BR, 16), lambda i: (0, i, 0)),
            pl.BlockSpec((BR, 16), lambda i: (i, 0)),
            pl.BlockSpec((BR, 16), lambda i: (i, 0)),
            pl.BlockSpec((1, 16), lambda i: (0, 0)),
        ],
        out_specs=[
            pl.BlockSpec((BR, 16), lambda i: (i, 0)),
            pl.BlockSpec((BR, 16), lambda i: (i, 0)),
        ],
        out_shape=[
            jax.ShapeDtypeStruct((NP, 16), jnp.float32),
            jax.ShapeDtypeStruct((NP, 16), jnp.float32),
        ],
    )(p, hs, dinv, bp)


def _pad2(w, r, c):
    return jnp.pad(w, ((0, r - w.shape[0]), (0, c - w.shape[1])))


def kernel(x, edge_index, W1, b1, W2, b2, W3, b3, W4, b4, W5, b5):
    pad = N + jnp.arange(EP - E, dtype=jnp.int32) % (NP - N)
    srcp = jnp.concatenate([edge_index[0], pad]).reshape(NCH, CW)
    dstp = jnp.concatenate([edge_index[1], pad]).reshape(NCH, CW)
    xp = jnp.pad(x, ((0, NP - N), (0, 0)))

    w1 = _pad2(W1, 256, 64)
    w2 = _pad2(W2, 64, 64)
    w3 = _pad2(W3, 64, 32)
    w4 = _pad2(W4, 32, 16)
    w5 = _pad2(W5, 16, 16)
    b1p = jnp.pad(b1, (0, 64 - 60)).reshape(1, 64)
    b2p = jnp.pad(b2, (0, 64 - 60)).reshape(1, 64)
    b3p = jnp.pad(b3, (0, 32 - 30)).reshape(1, 32)
    b4p = b4.reshape(1, 16)
    b5p = jnp.pad(b5, (0, 16 - 10)).reshape(1, 16)

    cnt = _sc_degree(dstp)
    dinv, hs1 = _tc_prep(xp, cnt, w1)

    p1 = _sc_aggregate(hs1, srcp, dstp)
    hs2 = _tc_mid(p1, hs1, dinv, b1p, w2)
    p2 = _sc_aggregate(hs2, srcp, dstp)
    hs3 = _tc_mid(p2, hs2, dinv, b2p, w3)
    p3 = _sc_aggregate(hs3, srcp, dstp)
    hs4 = _tc_mid(p3, hs3, dinv, b3p, w4)
    p4 = _sc_aggregate(hs4, srcp, dstp)
    hs5, emb = _tc_mid(p4, hs4, dinv, b4p, w5, emit_emb=True)
    p5 = _sc_aggregate(hs5, srcp, dstp)
    out_f, logp_f = _tc_final(p5, hs5, dinv, b5p)

    return out_f[:N, :10], logp_f[:N, :10], emb[:N, :]


# speedup vs baseline: 27.6158x; 1.0692x over previous
import functools

import jax
import jax.numpy as jnp
from jax import lax
from jax.experimental import pallas as pl
from jax.experimental.pallas import tpu as pltpu
from jax.experimental.pallas import tpu_sc as plsc

N = 10000
E = 160000
CW = 128
NCH = E // CW
NSC = 2
NTL = 16
NW = NSC * NTL
CB = NCH // NW
CMAX = CB + 1
RPT = N // NTL
BR = 2000
NBUF = 4


@functools.cache
def _sc_mesh():
    return plsc.VectorSubcoreMesh(core_axis_name="c", subcore_axis_name="s",
                                  num_cores=NSC, num_subcores=NTL)


def _tile_chunks(c, s):
    wid = c * NTL + s
    base = CB * wid + jnp.maximum(wid - (NW - 2), 0)
    nj = jnp.where(wid >= NW - 2, CMAX, CB)
    return base, nj


def _deg_body(dst_hbm, o0_hbm, o1_hbm, idx_v, buf_v, acc_sh, sem):
    c = lax.axis_index("c")
    s = lax.axis_index("s")
    base, nj = _tile_chunks(c, s)
    pltpu.sync_copy(dst_hbm.at[pl.ds(base, CMAX)], idx_v)
    for i in range(CW):
        buf_v[i, :] = jnp.zeros((16,), jnp.float32)
    for k in range(RPT // CW):
        pltpu.sync_copy(buf_v, acc_sh.at[pl.ds(s * RPT + k * CW, CW)])
    pltpu.sync_copy(buf_v.at[pl.ds(0, RPT % CW)],
                    acc_sh.at[pl.ds(s * RPT + (RPT // CW) * CW, RPT % CW)])
    for i in range(CW):
        buf_v[i, :] = jnp.full((16,), 1.0, jnp.float32)
    plsc.subcore_barrier()

    def chunk(j, carry):
        @pl.when(j < nj)
        def _():
            pltpu.sync_copy(buf_v, acc_sh.at[idx_v.at[j]], add=True)
        return carry

    lax.fori_loop(0, CMAX, chunk, 0)
    plsc.subcore_barrier()

    @pl.when(c == 0)
    def _():
        pltpu.sync_copy(acc_sh.at[pl.ds(s * RPT, RPT)],
                        o0_hbm.at[pl.ds(s * RPT, RPT)])

    @pl.when(c == 1)
    def _():
        pltpu.sync_copy(acc_sh.at[pl.ds(s * RPT, RPT)],
                        o1_hbm.at[pl.ds(s * RPT, RPT)])


def _sc_degree(dstp):
    return pl.kernel(
        _deg_body,
        out_type=[jax.ShapeDtypeStruct((N, 16), jnp.float32),
                  jax.ShapeDtypeStruct((N, 16), jnp.float32)],
        mesh=_sc_mesh(),
        scratch_types=[
            pltpu.VMEM((CMAX, CW), jnp.int32),
            pltpu.VMEM((CW, 16), jnp.float32),
            pltpu.VMEM_SHARED((N, 16), jnp.float32),
            pltpu.SemaphoreType.DMA,
        ],
        compiler_params=pltpu.CompilerParams(use_tc_tiling_on_sc=False),
    )(dstp)


def _agg_body(hs_hbm, src_hbm, dst_hbm, out_hbm, sidx_v, didx_v, row_v,
              acc_sh, sems, *, F):
    c = lax.axis_index("c")
    s = lax.axis_index("s")
    base, nj = _tile_chunks(c, s)
    pltpu.sync_copy(src_hbm.at[pl.ds(base, CMAX)], sidx_v)
    pltpu.sync_copy(dst_hbm.at[pl.ds(base, CMAX)], didx_v)
    for i in range(CW):
        for k in range(F // 16):
            row_v[0, i, pl.ds(k * 16, 16)] = jnp.zeros((16,), jnp.float32)
    for k in range(RPT // CW):
        pltpu.sync_copy(row_v.at[0], acc_sh.at[pl.ds(s * RPT + k * CW, CW)])
    pltpu.sync_copy(row_v.at[0, pl.ds(0, RPT % CW)],
                    acc_sh.at[pl.ds(s * RPT + (RPT // CW) * CW, RPT % CW)])
    plsc.subcore_barrier()

    for b in range(NBUF):
        pltpu.async_copy(hs_hbm.at[sidx_v.at[b]], row_v.at[b], sems.at[b])

    def ring(t, carry):
        for b in range(NBUF):
            j = t + b

            @pl.when(j < nj)
            def _():
                pltpu.make_async_copy(hs_hbm.at[sidx_v.at[j]], row_v.at[b],
                                      sems.at[b]).wait()
                pltpu.sync_copy(row_v.at[b], acc_sh.at[didx_v.at[j]],
                                add=True)

                @pl.when(j + NBUF < nj)
                def _():
                    pltpu.async_copy(hs_hbm.at[sidx_v.at[j + NBUF]],
                                     row_v.at[b], sems.at[b])
        return carry

    lax.fori_loop(0, (CMAX + NBUF - 1) // NBUF,
                  lambda t, cy: ring(t * NBUF, cy), 0)
    plsc.subcore_barrier()
    pltpu.sync_copy(acc_sh.at[pl.ds(s * RPT, RPT)],
                    out_hbm.at[c, pl.ds(s * RPT, RPT)])


def _sc_aggregate(hs, srcp, dstp):
    F = hs.shape[1]

    def body(*refs):
        _agg_body(*refs, F=F)

    return pl.kernel(
        body,
        out_type=jax.ShapeDtypeStruct((NSC, N, F), jnp.float32),
        mesh=_sc_mesh(),
        scratch_types=[
            pltpu.VMEM((CMAX, CW), jnp.int32),
            pltpu.VMEM((CMAX, CW), jnp.int32),
            pltpu.VMEM((NBUF, CW, F), jnp.float32),
            pltpu.VMEM_SHARED((N, F), jnp.float32),
            pltpu.SemaphoreType.DMA((NBUF,)),
        ],
        compiler_params=pltpu.CompilerParams(use_tc_tiling_on_sc=False),
    )(hs, srcp, dstp)


def _mm1_body(x_ref, w_ref, h_ref):
    h_ref[...] = jnp.dot(x_ref[...], w_ref[...],
                         preferred_element_type=jnp.float32)


def _tc_mm1(x, w1p):
    f = w1p.shape[1]
    return pl.pallas_call(
        _mm1_body,
        grid=(N // BR,),
        in_specs=[
            pl.BlockSpec((BR, 256), lambda i: (i, 0)),
            pl.BlockSpec((256, f), lambda i: (0, 0)),
        ],
        out_specs=pl.BlockSpec((BR, f), lambda i: (i, 0)),
        out_shape=jax.ShapeDtypeStruct((N, f), jnp.float32),
    )(x, w1p)


def _scale_body(h_ref, c0_ref, c1_ref, dinv_ref, hs_ref):
    deg = c0_ref[...] + c1_ref[...] + 1.0
    dinv = lax.rsqrt(deg)
    dinv_ref[...] = dinv
    hs_ref[...] = h_ref[...] * dinv[:, :1]


def _tc_prep(x, cnt0, cnt1, w1p):
    f = w1p.shape[1]
    h1 = _tc_mm1(x, w1p)
    return pl.pallas_call(
        _scale_body,
        grid=(N // BR,),
        in_specs=[
            pl.BlockSpec((BR, f), lambda i: (i, 0)),
            pl.BlockSpec((BR, 16), lambda i: (i, 0)),
            pl.BlockSpec((BR, 16), lambda i: (i, 0)),
        ],
        out_specs=[
            pl.BlockSpec((BR, 16), lambda i: (i, 0)),
            pl.BlockSpec((BR, f), lambda i: (i, 0)),
        ],
        out_shape=[
            jax.ShapeDtypeStruct((N, 16), jnp.float32),
            jax.ShapeDtypeStruct((N, f), jnp.float32),
        ],
    )(h1, cnt0, cnt1)


def _mid_body(p_ref, hs_ref, dinv_ref, b_ref, w_ref, hso_ref, emb_ref=None):
    d = dinv_ref[...][:, :1]
    o = d * (p_ref[0] + p_ref[1] + hs_ref[...]) + b_ref[...]
    xn = jnp.maximum(o, 0.0)
    if emb_ref is not None:
        emb_ref[...] = xn
    hso_ref[...] = jnp.dot(xn, w_ref[...],
                           preferred_element_type=jnp.float32) * d


def _tc_mid(p, hs, dinv, bp, wp, emit_emb=False):
    fi, fo = wp.shape
    outs = [jax.ShapeDtypeStruct((N, fo), jnp.float32)]
    out_specs = [pl.BlockSpec((BR, fo), lambda i: (i, 0))]
    if emit_emb:
        outs.append(jax.ShapeDtypeStruct((N, fi), jnp.float32))
        out_specs.append(pl.BlockSpec((BR, fi), lambda i: (i, 0)))
    res = pl.pallas_call(
        _mid_body,
        grid=(N // BR,),
        in_specs=[
            pl.BlockSpec((NSC, BR, fi), lambda i: (0, i, 0)),
            pl.BlockSpec((BR, fi), lambda i: (i, 0)),
            pl.BlockSpec((BR, 16), lambda i: (i, 0)),
            pl.BlockSpec((1, fi), lambda i: (0, 0)),
            pl.BlockSpec((fi, fo), lambda i: (0, 0)),
        ],
        out_specs=out_specs,
        out_shape=outs,
    )(p, hs, dinv, bp, wp)
    return res if emit_emb else res[0]


def _fin_body(p_ref, hs_ref, dinv_ref, b_ref, out_ref, logp_ref):
    d = dinv_ref[...][:, :1]
    o = d * (p_ref[0] + p_ref[1] + hs_ref[...]) + b_ref[...]
    col = lax.broadcasted_iota(jnp.int32, (BR, 16), 1)
    valid = col < 10
    om = jnp.where(valid, o, jnp.float32(-1e30))
    m = jnp.max(om, axis=1, keepdims=True)
    ex = jnp.where(valid, jnp.exp(o - m), 0.0)
    lse = jnp.log(jnp.sum(ex, axis=1, keepdims=True)) + m
    out_ref[...] = o[:, :10]
    logp_ref[...] = (o - lse)[:, :10]


def _tc_final(p, hs, dinv, bp):
    return pl.pallas_call(
        _fin_body,
        grid=(N // BR,),
        in_specs=[
            pl.BlockSpec((NSC, BR, 16), lambda i: (0, i, 0)),
            pl.BlockSpec((BR, 16), lambda i: (i, 0)),
            pl.BlockSpec((BR, 16), lambda i: (i, 0)),
            pl.BlockSpec((1, 16), lambda i: (0, 0)),
        ],
        out_specs=[
            pl.BlockSpec((BR, 10), lambda i: (i, 0)),
            pl.BlockSpec((BR, 10), lambda i: (i, 0)),
        ],
        out_shape=[
            jax.ShapeDtypeStruct((N, 10), jnp.float32),
            jax.ShapeDtypeStruct((N, 10), jnp.float32),
        ],
    )(p, hs, dinv, bp)


def _pad2(w, r, c):
    return jnp.pad(w, ((0, r - w.shape[0]), (0, c - w.shape[1])))


def kernel(x, edge_index, W1, b1, W2, b2, W3, b3, W4, b4, W5, b5):
    srcp = edge_index[0].reshape(NCH, CW)
    dstp = edge_index[1].reshape(NCH, CW)

    w1 = _pad2(W1, 256, 64)
    w2 = _pad2(W2, 64, 64)
    w3 = _pad2(W3, 64, 32)
    w4 = _pad2(W4, 32, 16)
    w5 = _pad2(W5, 16, 16)
    b1p = jnp.pad(b1, (0, 64 - 60)).reshape(1, 64)
    b2p = jnp.pad(b2, (0, 64 - 60)).reshape(1, 64)
    b3p = jnp.pad(b3, (0, 32 - 30)).reshape(1, 32)
    b4p = b4.reshape(1, 16)
    b5p = jnp.pad(b5, (0, 16 - 10)).reshape(1, 16)

    cnt0, cnt1 = _sc_degree(dstp)
    dinv, hs1 = _tc_prep(x, cnt0, cnt1, w1)

    p1 = _sc_aggregate(hs1, srcp, dstp)
    hs2 = _tc_mid(p1, hs1, dinv, b1p, w2)
    p2 = _sc_aggregate(hs2, srcp, dstp)
    hs3 = _tc_mid(p2, hs2, dinv, b2p, w3)
    p3 = _sc_aggregate(hs3, srcp, dstp)
    hs4 = _tc_mid(p3, hs3, dinv, b3p, w4)
    p4 = _sc_aggregate(hs4, srcp, dstp)
    hs5, emb = _tc_mid(p4, hs4, dinv, b4p, w5, emit_emb=True)
    p5 = _sc_aggregate(hs5, srcp, dstp)
    out, logp = _tc_final(p5, hs5, dinv, b5p)

    return out, logp, emb


# speedup vs baseline: 27.8695x; 1.0092x over previous
import functools

import jax
import jax.numpy as jnp
from jax import lax
from jax.experimental import pallas as pl
from jax.experimental.pallas import tpu as pltpu
from jax.experimental.pallas import tpu_sc as plsc

N = 10000
E = 160000
CW = 128
NCH = E // CW
NSC = 2
NTL = 16
NW = NSC * NTL
CB = NCH // NW
CMAX = CB + 1
RPT = N // NTL
BR = 5000
NBUF = 4


@functools.cache
def _sc_mesh():
    return plsc.VectorSubcoreMesh(core_axis_name="c", subcore_axis_name="s",
                                  num_cores=NSC, num_subcores=NTL)


def _tile_chunks(c, s):
    wid = c * NTL + s
    base = CB * wid + jnp.maximum(wid - (NW - 2), 0)
    nj = jnp.where(wid >= NW - 2, CMAX, CB)
    return base, nj


def _deg_body(dst_hbm, o0_hbm, o1_hbm, idx_v, buf_v, acc_sh, sem):
    c = lax.axis_index("c")
    s = lax.axis_index("s")
    base, nj = _tile_chunks(c, s)
    pltpu.sync_copy(dst_hbm.at[pl.ds(base, CMAX)], idx_v)
    for i in range(CW):
        buf_v[i, :] = jnp.zeros((16,), jnp.float32)
    for k in range(RPT // CW):
        pltpu.sync_copy(buf_v, acc_sh.at[pl.ds(s * RPT + k * CW, CW)])
    pltpu.sync_copy(buf_v.at[pl.ds(0, RPT % CW)],
                    acc_sh.at[pl.ds(s * RPT + (RPT // CW) * CW, RPT % CW)])
    for i in range(CW):
        buf_v[i, :] = jnp.full((16,), 1.0, jnp.float32)
    plsc.subcore_barrier()

    def chunk(j, carry):
        @pl.when(j < nj)
        def _():
            pltpu.sync_copy(buf_v, acc_sh.at[idx_v.at[j]], add=True)
        return carry

    lax.fori_loop(0, CMAX, chunk, 0)
    plsc.subcore_barrier()

    @pl.when(c == 0)
    def _():
        pltpu.sync_copy(acc_sh.at[pl.ds(s * RPT, RPT)],
                        o0_hbm.at[pl.ds(s * RPT, RPT)])

    @pl.when(c == 1)
    def _():
        pltpu.sync_copy(acc_sh.at[pl.ds(s * RPT, RPT)],
                        o1_hbm.at[pl.ds(s * RPT, RPT)])


def _sc_degree(dstp):
    return pl.kernel(
        _deg_body,
        out_type=[jax.ShapeDtypeStruct((N, 16), jnp.float32),
                  jax.ShapeDtypeStruct((N, 16), jnp.float32)],
        mesh=_sc_mesh(),
        scratch_types=[
            pltpu.VMEM((CMAX, CW), jnp.int32),
            pltpu.VMEM((CW, 16), jnp.float32),
            pltpu.VMEM_SHARED((N, 16), jnp.float32),
            pltpu.SemaphoreType.DMA,
        ],
        compiler_params=pltpu.CompilerParams(use_tc_tiling_on_sc=False),
    )(dstp)


def _agg_body(hs_hbm, src_hbm, dst_hbm, out_hbm, sidx_v, didx_v, row_v,
              acc_sh, sems, *, F):
    c = lax.axis_index("c")
    s = lax.axis_index("s")
    base, nj = _tile_chunks(c, s)
    pltpu.sync_copy(src_hbm.at[pl.ds(base, CMAX)], sidx_v)
    pltpu.sync_copy(dst_hbm.at[pl.ds(base, CMAX)], didx_v)
    for i in range(CW):
        for k in range(F // 16):
            row_v[0, i, pl.ds(k * 16, 16)] = jnp.zeros((16,), jnp.float32)
    for k in range(RPT // CW):
        pltpu.sync_copy(row_v.at[0], acc_sh.at[pl.ds(s * RPT + k * CW, CW)])
    pltpu.sync_copy(row_v.at[0, pl.ds(0, RPT % CW)],
                    acc_sh.at[pl.ds(s * RPT + (RPT // CW) * CW, RPT % CW)])
    plsc.subcore_barrier()

    for b in range(NBUF):
        pltpu.async_copy(hs_hbm.at[sidx_v.at[b]], row_v.at[b], sems.at[b])

    def ring(t, carry):
        for b in range(NBUF):
            j = t + b

            @pl.when(j < nj)
            def _():
                pltpu.make_async_copy(hs_hbm.at[sidx_v.at[j]], row_v.at[b],
                                      sems.at[b]).wait()
                pltpu.sync_copy(row_v.at[b], acc_sh.at[didx_v.at[j]],
                                add=True)

                @pl.when(j + NBUF < nj)
                def _():
                    pltpu.async_copy(hs_hbm.at[sidx_v.at[j + NBUF]],
                                     row_v.at[b], sems.at[b])
        return carry

    lax.fori_loop(0, (CMAX + NBUF - 1) // NBUF,
                  lambda t, cy: ring(t * NBUF, cy), 0)
    plsc.subcore_barrier()
    pltpu.sync_copy(acc_sh.at[pl.ds(s * RPT, RPT)],
                    out_hbm.at[c, pl.ds(s * RPT, RPT)])


def _sc_aggregate(hs, srcp, dstp):
    F = hs.shape[1]

    def body(*refs):
        _agg_body(*refs, F=F)

    return pl.kernel(
        body,
        out_type=jax.ShapeDtypeStruct((NSC, N, F), jnp.float32),
        mesh=_sc_mesh(),
        scratch_types=[
            pltpu.VMEM((CMAX, CW), jnp.int32),
            pltpu.VMEM((CMAX, CW), jnp.int32),
            pltpu.VMEM((NBUF, CW, F), jnp.float32),
            pltpu.VMEM_SHARED((N, F), jnp.float32),
            pltpu.SemaphoreType.DMA((NBUF,)),
        ],
        compiler_params=pltpu.CompilerParams(use_tc_tiling_on_sc=False),
    )(hs, srcp, dstp)


def _mm1_body(x_ref, w_ref, h_ref):
    h_ref[...] = jnp.dot(x_ref[...], w_ref[...],
                         preferred_element_type=jnp.float32)


def _tc_mm1(x, w1p):
    f = w1p.shape[1]
    return pl.pallas_call(
        _mm1_body,
        grid=(N // BR,),
        in_specs=[
            pl.BlockSpec((BR, 256), lambda i: (i, 0)),
            pl.BlockSpec((256, f), lambda i: (0, 0)),
        ],
        out_specs=pl.BlockSpec((BR, f), lambda i: (i, 0)),
        out_shape=jax.ShapeDtypeStruct((N, f), jnp.float32),
    )(x, w1p)


def _scale_body(h_ref, c0_ref, c1_ref, dinv_ref, hs_ref):
    deg = c0_ref[...] + c1_ref[...] + 1.0
    dinv = lax.rsqrt(deg)
    dinv_ref[...] = dinv
    hs_ref[...] = h_ref[...] * dinv[:, :1]


def _tc_prep(x, cnt0, cnt1, w1p):
    f = w1p.shape[1]
    h1 = _tc_mm1(x, w1p)
    return pl.pallas_call(
        _scale_body,
        grid=(N // BR,),
        in_specs=[
            pl.BlockSpec((BR, f), lambda i: (i, 0)),
            pl.BlockSpec((BR, 16), lambda i: (i, 0)),
            pl.BlockSpec((BR, 16), lambda i: (i, 0)),
        ],
        out_specs=[
            pl.BlockSpec((BR, 16), lambda i: (i, 0)),
            pl.BlockSpec((BR, f), lambda i: (i, 0)),
        ],
        out_shape=[
            jax.ShapeDtypeStruct((N, 16), jnp.float32),
            jax.ShapeDtypeStruct((N, f), jnp.float32),
        ],
    )(h1, cnt0, cnt1)


def _mid_body(p_ref, hs_ref, dinv_ref, b_ref, w_ref, hso_ref, emb_ref=None):
    d = dinv_ref[...][:, :1]
    o = d * (p_ref[0] + p_ref[1] + hs_ref[...]) + b_ref[...]
    xn = jnp.maximum(o, 0.0)
    if emb_ref is not None:
        emb_ref[...] = xn
    hso_ref[...] = jnp.dot(xn, w_ref[...],
                           preferred_element_type=jnp.float32) * d


def _tc_mid(p, hs, dinv, bp, wp, emit_emb=False):
    fi, fo = wp.shape
    outs = [jax.ShapeDtypeStruct((N, fo), jnp.float32)]
    out_specs = [pl.BlockSpec((BR, fo), lambda i: (i, 0))]
    if emit_emb:
        outs.append(jax.ShapeDtypeStruct((N, fi), jnp.float32))
        out_specs.append(pl.BlockSpec((BR, fi), lambda i: (i, 0)))
    res = pl.pallas_call(
        _mid_body,
        grid=(N // BR,),
        in_specs=[
            pl.BlockSpec((NSC, BR, fi), lambda i: (0, i, 0)),
            pl.BlockSpec((BR, fi), lambda i: (i, 0)),
            pl.BlockSpec((BR, 16), lambda i: (i, 0)),
            pl.BlockSpec((1, fi), lambda i: (0, 0)),
            pl.BlockSpec((fi, fo), lambda i: (0, 0)),
        ],
        out_specs=out_specs,
        out_shape=outs,
    )(p, hs, dinv, bp, wp)
    return res if emit_emb else res[0]


def _fin_body(p_ref, hs_ref, dinv_ref, b_ref, out_ref, logp_ref):
    d = dinv_ref[...][:, :1]
    o = d * (p_ref[0] + p_ref[1] + hs_ref[...]) + b_ref[...]
    col = lax.broadcasted_iota(jnp.int32, (BR, 16), 1)
    valid = col < 10
    om = jnp.where(valid, o, jnp.float32(-1e30))
    m = jnp.max(om, axis=1, keepdims=True)
    ex = jnp.where(valid, jnp.exp(o - m), 0.0)
    lse = jnp.log(jnp.sum(ex, axis=1, keepdims=True)) + m
    out_ref[...] = o[:, :10]
    logp_ref[...] = (o - lse)[:, :10]


def _tc_final(p, hs, dinv, bp):
    return pl.pallas_call(
        _fin_body,
        grid=(N // BR,),
        in_specs=[
            pl.BlockSpec((NSC, BR, 16), lambda i: (0, i, 0)),
            pl.BlockSpec((BR, 16), lambda i: (i, 0)),
            pl.BlockSpec((BR, 16), lambda i: (i, 0)),
            pl.BlockSpec((1, 16), lambda i: (0, 0)),
        ],
        out_specs=[
            pl.BlockSpec((BR, 10), lambda i: (i, 0)),
            pl.BlockSpec((BR, 10), lambda i: (i, 0)),
        ],
        out_shape=[
            jax.ShapeDtypeStruct((N, 10), jnp.float32),
            jax.ShapeDtypeStruct((N, 10), jnp.float32),
        ],
    )(p, hs, dinv, bp)


def _pad2(w, r, c):
    return jnp.pad(w, ((0, r - w.shape[0]), (0, c - w.shape[1])))


def kernel(x, edge_index, W1, b1, W2, b2, W3, b3, W4, b4, W5, b5):
    srcp = edge_index[0].reshape(NCH, CW)
    dstp = edge_index[1].reshape(NCH, CW)

    w1 = _pad2(W1, 256, 64)
    w2 = _pad2(W2, 64, 64)
    w3 = _pad2(W3, 64, 32)
    w4 = _pad2(W4, 32, 16)
    w5 = _pad2(W5, 16, 16)
    b1p = jnp.pad(b1, (0, 64 - 60)).reshape(1, 64)
    b2p = jnp.pad(b2, (0, 64 - 60)).reshape(1, 64)
    b3p = jnp.pad(b3, (0, 32 - 30)).reshape(1, 32)
    b4p = b4.reshape(1, 16)
    b5p = jnp.pad(b5, (0, 16 - 10)).reshape(1, 16)

    cnt0, cnt1 = _sc_degree(dstp)
    dinv, hs1 = _tc_prep(x, cnt0, cnt1, w1)

    p1 = _sc_aggregate(hs1, srcp, dstp)
    hs2 = _tc_mid(p1, hs1, dinv, b1p, w2)
    p2 = _sc_aggregate(hs2, srcp, dstp)
    hs3 = _tc_mid(p2, hs2, dinv, b2p, w3)
    p3 = _sc_aggregate(hs3, srcp, dstp)
    hs4 = _tc_mid(p3, hs3, dinv, b3p, w4)
    p4 = _sc_aggregate(hs4, srcp, dstp)
    hs5, emb = _tc_mid(p4, hs4, dinv, b4p, w5, emit_emb=True)
    p5 = _sc_aggregate(hs5, srcp, dstp)
    out, logp = _tc_final(p5, hs5, dinv, b5p)

    return out, logp, emb


# speedup vs baseline: 28.2995x; 1.0154x over previous
import functools

import jax
import jax.numpy as jnp
from jax import lax
from jax.experimental import pallas as pl
from jax.experimental.pallas import tpu as pltpu
from jax.experimental.pallas import tpu_sc as plsc

N = 10000
E = 160000
CW = 128
NCH = E // CW
NSC = 2
NTL = 16
NW = NSC * NTL
CB = NCH // NW
CMAX = CB + 1
RPT = N // NTL
BR = 5000
NBUF = 6


@functools.cache
def _sc_mesh():
    return plsc.VectorSubcoreMesh(core_axis_name="c", subcore_axis_name="s",
                                  num_cores=NSC, num_subcores=NTL)


def _tile_chunks(c, s):
    wid = c * NTL + s
    base = CB * wid + jnp.maximum(wid - (NW - 2), 0)
    nj = jnp.where(wid >= NW - 2, CMAX, CB)
    return base, nj


def _deg_body(dst_hbm, o0_hbm, o1_hbm, idx_v, buf_v, acc_sh, sem):
    c = lax.axis_index("c")
    s = lax.axis_index("s")
    base, nj = _tile_chunks(c, s)
    pltpu.sync_copy(dst_hbm.at[pl.ds(base, CMAX)], idx_v)
    for i in range(CW):
        buf_v[i, :] = jnp.zeros((16,), jnp.float32)
    for k in range(RPT // CW):
        pltpu.sync_copy(buf_v, acc_sh.at[pl.ds(s * RPT + k * CW, CW)])
    pltpu.sync_copy(buf_v.at[pl.ds(0, RPT % CW)],
                    acc_sh.at[pl.ds(s * RPT + (RPT // CW) * CW, RPT % CW)])
    for i in range(CW):
        buf_v[i, :] = jnp.full((16,), 1.0, jnp.float32)
    plsc.subcore_barrier()

    def chunk(j, carry):
        @pl.when(j < nj)
        def _():
            pltpu.sync_copy(buf_v, acc_sh.at[idx_v.at[j]], add=True)
        return carry

    lax.fori_loop(0, CMAX, chunk, 0)
    plsc.subcore_barrier()

    @pl.when(c == 0)
    def _():
        pltpu.sync_copy(acc_sh.at[pl.ds(s * RPT, RPT)],
                        o0_hbm.at[pl.ds(s * RPT, RPT)])

    @pl.when(c == 1)
    def _():
        pltpu.sync_copy(acc_sh.at[pl.ds(s * RPT, RPT)],
                        o1_hbm.at[pl.ds(s * RPT, RPT)])


def _sc_degree(dstp):
    return pl.kernel(
        _deg_body,
        out_type=[jax.ShapeDtypeStruct((N, 16), jnp.float32),
                  jax.ShapeDtypeStruct((N, 16), jnp.float32)],
        mesh=_sc_mesh(),
        scratch_types=[
            pltpu.VMEM((CMAX, CW), jnp.int32),
            pltpu.VMEM((CW, 16), jnp.float32),
            pltpu.VMEM_SHARED((N, 16), jnp.float32),
            pltpu.SemaphoreType.DMA,
        ],
        compiler_params=pltpu.CompilerParams(use_tc_tiling_on_sc=False),
    )(dstp)


def _agg_body(hs_hbm, src_hbm, dst_hbm, out_hbm, sidx_v, didx_v, row_v,
              acc_sh, sems, *, F):
    c = lax.axis_index("c")
    s = lax.axis_index("s")
    base, nj = _tile_chunks(c, s)
    pltpu.sync_copy(src_hbm.at[pl.ds(base, CMAX)], sidx_v)
    pltpu.sync_copy(dst_hbm.at[pl.ds(base, CMAX)], didx_v)
    for i in range(CW):
        for k in range(F // 16):
            row_v[0, i, pl.ds(k * 16, 16)] = jnp.zeros((16,), jnp.float32)
    for k in range(RPT // CW):
        pltpu.sync_copy(row_v.at[0], acc_sh.at[pl.ds(s * RPT + k * CW, CW)])
    pltpu.sync_copy(row_v.at[0, pl.ds(0, RPT % CW)],
                    acc_sh.at[pl.ds(s * RPT + (RPT // CW) * CW, RPT % CW)])
    plsc.subcore_barrier()

    for b in range(NBUF):
        pltpu.async_copy(hs_hbm.at[sidx_v.at[b]], row_v.at[b], sems.at[b])

    def ring(t, carry):
        for b in range(NBUF):
            j = t + b

            @pl.when(j < nj)
            def _():
                pltpu.make_async_copy(hs_hbm.at[sidx_v.at[j]], row_v.at[b],
                                      sems.at[b]).wait()
                pltpu.sync_copy(row_v.at[b], acc_sh.at[didx_v.at[j]],
                                add=True)

                @pl.when(j + NBUF < nj)
                def _():
                    pltpu.async_copy(hs_hbm.at[sidx_v.at[j + NBUF]],
                                     row_v.at[b], sems.at[b])
        return carry

    lax.fori_loop(0, (CMAX + NBUF - 1) // NBUF,
                  lambda t, cy: ring(t * NBUF, cy), 0)
    plsc.subcore_barrier()
    pltpu.sync_copy(acc_sh.at[pl.ds(s * RPT, RPT)],
                    out_hbm.at[c, pl.ds(s * RPT, RPT)])


def _sc_aggregate(hs, srcp, dstp):
    F = hs.shape[1]

    def body(*refs):
        _agg_body(*refs, F=F)

    return pl.kernel(
        body,
        out_type=jax.ShapeDtypeStruct((NSC, N, F), jnp.float32),
        mesh=_sc_mesh(),
        scratch_types=[
            pltpu.VMEM((CMAX, CW), jnp.int32),
            pltpu.VMEM((CMAX, CW), jnp.int32),
            pltpu.VMEM((NBUF, CW, F), jnp.float32),
            pltpu.VMEM_SHARED((N, F), jnp.float32),
            pltpu.SemaphoreType.DMA((NBUF,)),
        ],
        compiler_params=pltpu.CompilerParams(use_tc_tiling_on_sc=False),
    )(hs, srcp, dstp)


def _prep_body(x_ref, c0_ref, c1_ref, w_ref, dinv_ref, hs_ref):
    deg = c0_ref[...] + c1_ref[...] + 1.0
    dinv = lax.rsqrt(deg)
    dinv_ref[...] = dinv
    h = jnp.dot(x_ref[...], w_ref[...], preferred_element_type=jnp.float32)
    hs_ref[...] = h * dinv[:, :1]


def _tc_prep(x, cnt0, cnt1, w1p):
    f = w1p.shape[1]
    return pl.pallas_call(
        _prep_body,
        grid=(N // BR,),
        in_specs=[
            pl.BlockSpec((BR, 256), lambda i: (i, 0)),
            pl.BlockSpec((BR, 16), lambda i: (i, 0)),
            pl.BlockSpec((BR, 16), lambda i: (i, 0)),
            pl.BlockSpec((256, f), lambda i: (0, 0)),
        ],
        out_specs=[
            pl.BlockSpec((BR, 16), lambda i: (i, 0)),
            pl.BlockSpec((BR, f), lambda i: (i, 0)),
        ],
        out_shape=[
            jax.ShapeDtypeStruct((N, 16), jnp.float32),
            jax.ShapeDtypeStruct((N, f), jnp.float32),
        ],
    )(x, cnt0, cnt1, w1p)


def _mid_body(p_ref, hs_ref, dinv_ref, b_ref, w_ref, hso_ref, emb_ref=None):
    d = dinv_ref[...][:, :1]
    o = d * (p_ref[0] + p_ref[1] + hs_ref[...]) + b_ref[...]
    xn = jnp.maximum(o, 0.0)
    if emb_ref is not None:
        emb_ref[...] = xn
    hso_ref[...] = jnp.dot(xn, w_ref[...],
                           preferred_element_type=jnp.float32) * d


def _tc_mid(p, hs, dinv, bp, wp, emit_emb=False):
    fi, fo = wp.shape
    outs = [jax.ShapeDtypeStruct((N, fo), jnp.float32)]
    out_specs = [pl.BlockSpec((BR, fo), lambda i: (i, 0))]
    if emit_emb:
        outs.append(jax.ShapeDtypeStruct((N, fi), jnp.float32))
        out_specs.append(pl.BlockSpec((BR, fi), lambda i: (i, 0)))
    res = pl.pallas_call(
        _mid_body,
        grid=(N // BR,),
        in_specs=[
            pl.BlockSpec((NSC, BR, fi), lambda i: (0, i, 0)),
            pl.BlockSpec((BR, fi), lambda i: (i, 0)),
            pl.BlockSpec((BR, 16), lambda i: (i, 0)),
            pl.BlockSpec((1, fi), lambda i: (0, 0)),
            pl.BlockSpec((fi, fo), lambda i: (0, 0)),
        ],
        out_specs=out_specs,
        out_shape=outs,
    )(p, hs, dinv, bp, wp)
    return res if emit_emb else res[0]


def _fin_body(p_ref, hs_ref, dinv_ref, b_ref, out_ref, logp_ref):
    d = dinv_ref[...][:, :1]
    o = d * (p_ref[0] + p_ref[1] + hs_ref[...]) + b_ref[...]
    col = lax.broadcasted_iota(jnp.int32, (BR, 16), 1)
    valid = col < 10
    om = jnp.where(valid, o, jnp.float32(-1e30))
    m = jnp.max(om, axis=1, keepdims=True)
    ex = jnp.where(valid, jnp.exp(o - m), 0.0)
    lse = jnp.log(jnp.sum(ex, axis=1, keepdims=True)) + m
    out_ref[...] = o[:, :10]
    logp_ref[...] = (o - lse)[:, :10]


def _tc_final(p, hs, dinv, bp):
    return pl.pallas_call(
        _fin_body,
        grid=(N // BR,),
        in_specs=[
            pl.BlockSpec((NSC, BR, 16), lambda i: (0, i, 0)),
            pl.BlockSpec((BR, 16), lambda i: (i, 0)),
            pl.BlockSpec((BR, 16), lambda i: (i, 0)),
            pl.BlockSpec((1, 16), lambda i: (0, 0)),
        ],
        out_specs=[
            pl.BlockSpec((BR, 10), lambda i: (i, 0)),
            pl.BlockSpec((BR, 10), lambda i: (i, 0)),
        ],
        out_shape=[
            jax.ShapeDtypeStruct((N, 10), jnp.float32),
            jax.ShapeDtypeStruct((N, 10), jnp.float32),
        ],
    )(p, hs, dinv, bp)


def _pad2(w, r, c):
    return jnp.pad(w, ((0, r - w.shape[0]), (0, c - w.shape[1])))


def kernel(x, edge_index, W1, b1, W2, b2, W3, b3, W4, b4, W5, b5):
    srcp = edge_index[0].reshape(NCH, CW)
    dstp = edge_index[1].reshape(NCH, CW)

    w1 = _pad2(W1, 256, 64)
    w2 = _pad2(W2, 64, 64)
    w3 = _pad2(W3, 64, 32)
    w4 = _pad2(W4, 32, 16)
    w5 = _pad2(W5, 16, 16)
    b1p = jnp.pad(b1, (0, 64 - 60)).reshape(1, 64)
    b2p = jnp.pad(b2, (0, 64 - 60)).reshape(1, 64)
    b3p = jnp.pad(b3, (0, 32 - 30)).reshape(1, 32)
    b4p = b4.reshape(1, 16)
    b5p = jnp.pad(b5, (0, 16 - 10)).reshape(1, 16)

    cnt0, cnt1 = _sc_degree(dstp)
    dinv, hs1 = _tc_prep(x, cnt0, cnt1, w1)

    p1 = _sc_aggregate(hs1, srcp, dstp)
    hs2 = _tc_mid(p1, hs1, dinv, b1p, w2)
    p2 = _sc_aggregate(hs2, srcp, dstp)
    hs3 = _tc_mid(p2, hs2, dinv, b2p, w3)
    p3 = _sc_aggregate(hs3, srcp, dstp)
    hs4 = _tc_mid(p3, hs3, dinv, b3p, w4)
    p4 = _sc_aggregate(hs4, srcp, dstp)
    hs5, emb = _tc_mid(p4, hs4, dinv, b4p, w5, emit_emb=True)
    p5 = _sc_aggregate(hs5, srcp, dstp)
    out, logp = _tc_final(p5, hs5, dinv, b5p)

    return out, logp, emb
